# Initial kernel scaffold; baseline (speedup 1.0000x reference)
import jax
import jax.numpy as jnp
from jax.experimental import pallas as pl


def kernel(s, v, edge_index, edge_attr, rbf, W1, b1, W2, b2, W3, b3, W4, b4):
    raise NotImplementedError("write your pallas kernel here")


# speedup vs baseline: 9.5518x; 9.5518x over previous
import jax
import jax.numpy as jnp
from jax import lax
from jax.experimental import pallas as pl
from jax.experimental.pallas import tpu as pltpu
from jax.experimental.pallas import tpu_sc as plsc

N = 10000
E = 320000
H = 128
NCORE = 2
NSUB = 16
LANES = 16
HC = 16
NCHUNK = H // HC
CPC = NCHUNK // NCORE
B = 80
TPE = E // NSUB
NB = TPE // B
NROWS = 624
NTAIL = N - NROWS * NSUB
EB = 2560
BN = 1000


def _filter_body(rbf_ref, w1_ref, b1_ref, w2_ref, b2_ref, out_ref):
    x = rbf_ref[...]
    hmid = jnp.dot(x, w1_ref[...].T, preferred_element_type=jnp.float32)
    hmid = hmid + b1_ref[...]
    hmid = hmid * jax.nn.sigmoid(hmid)
    f = jnp.dot(hmid, w2_ref[...].T, preferred_element_type=jnp.float32)
    f = f + b2_ref[...]
    for c in range(NCHUNK):
        out_ref[c] = f[:, 2 * HC * c:2 * HC * (c + 1)]


def _filter_net(rbf, W1, b1, W2p, b2p):
    grid = (E // EB,)
    return pl.pallas_call(
        _filter_body,
        grid=grid,
        in_specs=[
            pl.BlockSpec((EB, 20), lambda i: (i, 0)),
            pl.BlockSpec((H, 20), lambda i: (0, 0)),
            pl.BlockSpec((1, H), lambda i: (0, 0)),
            pl.BlockSpec((2 * H, H), lambda i: (0, 0)),
            pl.BlockSpec((1, 2 * H), lambda i: (0, 0)),
        ],
        out_specs=pl.BlockSpec((NCHUNK, EB, 2 * HC), lambda i: (0, i, 0)),
        out_shape=jax.ShapeDtypeStruct((NCHUNK, E, 2 * HC), jnp.float32),
    )(rbf, W1, b1.reshape(1, H), W2p, b2p.reshape(1, 2 * H))


def _sc_body(s4, vt4, filtc, col2, row2, ea, zer16, zer48,
             ms_out, mv_out,
             acc_ms, acc_mv, colv, rowv, fb, ub0, ub1, sb, vb, msb, mvb,
             sem_in, sem_out):
    ubs = (ub0, ub1)
    core = lax.axis_index("c")
    sub = lax.axis_index("s")
    rbase = sub * NROWS
    ebase = sub * TPE

    pltpu.sync_copy(col2.at[sub], colv)
    pltpu.sync_copy(row2.at[sub], rowv)

    def issue_inputs(sl, bi, chunk):
        e0 = ebase + bi * B
        pltpu.async_copy(filtc.at[chunk, pl.ds(e0, B)], fb.at[sl], sem_in.at[sl])
        pltpu.async_copy(ea.at[pl.ds(e0 * 4, B * 4)],
                         ubs[sl].at[pl.ds(0, B * 4)], sem_in.at[sl])
        pltpu.async_copy(s4.at[chunk].at[colv.at[bi]], sb.at[sl], sem_in.at[sl])
        pltpu.async_copy(vt4.at[chunk].at[colv.at[bi]], vb.at[sl], sem_in.at[sl])

    def wait_inputs(sl):
        pltpu.make_async_copy(filtc.at[0, pl.ds(0, B)], fb.at[sl], sem_in.at[sl]).wait()
        pltpu.make_async_copy(ea.at[pl.ds(0, B * 4)],
                              ubs[sl].at[pl.ds(0, B * 4)], sem_in.at[sl]).wait()
        pltpu.make_async_copy(zer16.at[pl.ds(0, B)], sb.at[sl], sem_in.at[sl]).wait()
        pltpu.make_async_copy(zer48.at[pl.ds(0, B)], vb.at[sl], sem_in.at[sl]).wait()

    def issue_scatter(sl, bi):
        pltpu.async_copy(msb.at[sl], acc_ms.at[rowv.at[bi]], sem_out.at[sl], add=True)
        pltpu.async_copy(mvb.at[sl], acc_mv.at[rowv.at[bi]], sem_out.at[sl], add=True)

    def wait_scatter(sl):
        pltpu.make_async_copy(zer16.at[pl.ds(0, B)], msb.at[sl], sem_out.at[sl]).wait()
        pltpu.make_async_copy(zer48.at[pl.ds(0, B)], mvb.at[sl], sem_out.at[sl]).wait()

    def compute(sl):
        uref = ubs[sl]

        def ebody(b2, carry):
            va = uref[pl.ds(b2 * 4, LANES)]
            fs = fb[sl, b2, pl.ds(0, LANES)]
            fv = fb[sl, b2, pl.ds(HC, LANES)]
            sc_ = sb[sl, b2, pl.ds(0, LANES)]
            msb[sl, b2, pl.ds(0, LANES)] = fs * sc_
            t = fv * sc_
            for k in range(3):
                vc = vb[sl, b2, pl.ds(HC * k, LANES)]
                uk = jnp.full((LANES,), va[k + 1])
                mvb[sl, b2, pl.ds(HC * k, LANES)] = fv * vc + t * uk
            return carry

        lax.fori_loop(0, B, ebody, 0)

    for cc in range(CPC):
        chunk = core * CPC + cc

        pltpu.sync_copy(zer16.at[pl.ds(rbase, NROWS)],
                        acc_ms.at[pl.ds(rbase, NROWS)])
        pltpu.sync_copy(zer48.at[pl.ds(rbase, NROWS)],
                        acc_mv.at[pl.ds(rbase, NROWS)])

        @pl.when(sub == NSUB - 1)
        def _zero_tail():
            pltpu.sync_copy(zer16.at[pl.ds(NROWS * NSUB, NTAIL)],
                            acc_ms.at[pl.ds(NROWS * NSUB, NTAIL)])
            pltpu.sync_copy(zer48.at[pl.ds(NROWS * NSUB, NTAIL)],
                            acc_mv.at[pl.ds(NROWS * NSUB, NTAIL)])

        for sl in range(2):
            pltpu.sync_copy(zer16.at[pl.ds(0, B)], msb.at[sl])
            pltpu.sync_copy(zer48.at[pl.ds(0, B)], mvb.at[sl])
        plsc.subcore_barrier()

        for sl in range(2):
            issue_scatter(sl, jnp.int32(0))
            issue_inputs(sl, jnp.int32(sl), chunk)

        def pair_body(bp, carry):
            for sl in range(2):
                bi = bp * 2 + sl
                wait_inputs(sl)
                wait_scatter(sl)
                compute(sl)
                issue_scatter(sl, bi)
                issue_inputs(sl, jnp.minimum(bi + 2, NB - 1), chunk)
            return carry

        lax.fori_loop(0, NB // 2, pair_body, 0)

        for sl in range(2):
            wait_inputs(sl)
            wait_scatter(sl)
        plsc.subcore_barrier()

        pltpu.sync_copy(acc_ms.at[pl.ds(rbase, NROWS)],
                        ms_out.at[chunk, pl.ds(rbase, NROWS)])
        pltpu.sync_copy(acc_mv.at[pl.ds(rbase, NROWS)],
                        mv_out.at[chunk, pl.ds(rbase, NROWS)])

        @pl.when(sub == NSUB - 1)
        def _flush_tail():
            pltpu.sync_copy(acc_ms.at[pl.ds(NROWS * NSUB, NTAIL)],
                            ms_out.at[chunk, pl.ds(NROWS * NSUB, NTAIL)])
            pltpu.sync_copy(acc_mv.at[pl.ds(NROWS * NSUB, NTAIL)],
                            mv_out.at[chunk, pl.ds(NROWS * NSUB, NTAIL)])

        plsc.subcore_barrier()


def _message_pass(s4, vt4, filtc, col2, row2, ea, zer16, zer48):
    mesh = plsc.VectorSubcoreMesh(core_axis_name="c", subcore_axis_name="s",
                                  num_cores=NCORE, num_subcores=NSUB)
    f32 = jnp.float32
    return pl.kernel(
        _sc_body,
        compiler_params=pltpu.CompilerParams(use_tc_tiling_on_sc=False),
        out_type=(jax.ShapeDtypeStruct((NCHUNK, N, HC), f32),
                  jax.ShapeDtypeStruct((NCHUNK, N, 3 * HC), f32)),
        mesh=mesh,
        scratch_types=[
            pltpu.VMEM_SHARED((N, HC), f32),
            pltpu.VMEM_SHARED((N, 3 * HC), f32),
            pltpu.VMEM((NB, B), jnp.int32),
            pltpu.VMEM((NB, B), jnp.int32),
            pltpu.VMEM((2, B, 2 * HC), f32),
            pltpu.VMEM((B * 4 + LANES,), f32),
            pltpu.VMEM((B * 4 + LANES,), f32),
            pltpu.VMEM((2, B, HC), f32),
            pltpu.VMEM((2, B, 3 * HC), f32),
            pltpu.VMEM((2, B, HC), f32),
            pltpu.VMEM((2, B, 3 * HC), f32),
            pltpu.SemaphoreType.DMA((2,)),
            pltpu.SemaphoreType.DMA((2,)),
        ],
    )(s4, vt4, filtc, col2, row2, ea, zer16, zer48)


def _update_body(s_ref, ms_ref, mv_ref, v_ref, w3_ref, b3_ref, w4_ref, b4_ref,
                 snew_ref, vnew_ref):
    s = s_ref[...]
    ms = ms_ref[...]
    mv = mv_ref[...]
    vn = jnp.sqrt(mv[:, 0, :] ** 2 + mv[:, 1, :] ** 2 + mv[:, 2, :] ** 2)
    x = jnp.concatenate([s, ms, vn], axis=1)
    hmid = jnp.dot(x, w3_ref[...].T, preferred_element_type=jnp.float32)
    hmid = hmid + b3_ref[...]
    hmid = hmid * jax.nn.sigmoid(hmid)
    u = jnp.dot(hmid, w4_ref[...].T, preferred_element_type=jnp.float32)
    u = u + b4_ref[...]
    delta_s = u[:, :H]
    alpha = u[:, H:2 * H]
    beta = u[:, 2 * H:]
    snew_ref[...] = s + delta_s
    vnew_ref[...] = alpha[:, None, :] * v_ref[...] + beta[:, None, :] * mv


def _node_update(s, msg_s, mv3, v3, W3, b3, W4, b4):
    grid = (N // BN,)
    return pl.pallas_call(
        _update_body,
        grid=grid,
        in_specs=[
            pl.BlockSpec((BN, H), lambda i: (i, 0)),
            pl.BlockSpec((BN, H), lambda i: (i, 0)),
            pl.BlockSpec((BN, 3, H), lambda i: (i, 0, 0)),
            pl.BlockSpec((BN, 3, H), lambda i: (i, 0, 0)),
            pl.BlockSpec((H, 3 * H), lambda i: (0, 0)),
            pl.BlockSpec((1, H), lambda i: (0, 0)),
            pl.BlockSpec((3 * H, H), lambda i: (0, 0)),
            pl.BlockSpec((1, 3 * H), lambda i: (0, 0)),
        ],
        out_specs=[
            pl.BlockSpec((BN, H), lambda i: (i, 0)),
            pl.BlockSpec((BN, 3, H), lambda i: (i, 0, 0)),
        ],
        out_shape=[
            jax.ShapeDtypeStruct((N, H), jnp.float32),
            jax.ShapeDtypeStruct((N, 3, H), jnp.float32),
        ],
    )(s, msg_s, mv3, v3, W3, b3.reshape(1, H), W4, b4.reshape(1, 3 * H))


def kernel(s, v, edge_index, edge_attr, rbf, W1, b1, W2, b2, W3, b3, W4, b4):
    row = edge_index[0]
    col = edge_index[1]
    col2 = col.reshape(NSUB, NB, B)
    row2 = row.reshape(NSUB, NB, B)

    s4 = s.reshape(N, NCHUNK, HC).transpose(1, 0, 2)
    vt4 = v.transpose(0, 2, 1).reshape(N, 3, NCHUNK, HC).transpose(2, 0, 1, 3)
    vt4 = vt4.reshape(NCHUNK, N, 3 * HC)
    zer16 = jnp.zeros((N, HC), jnp.float32)
    zer48 = jnp.zeros((N, 3 * HC), jnp.float32)

    perm = jnp.array(
        [(HC * c + i) if i < HC else (H + HC * c + i - HC)
         for c in range(NCHUNK) for i in range(2 * HC)], dtype=jnp.int32)
    W2p = W2[perm]
    b2p = b2[perm]

    filtc = _filter_net(rbf, W1, b1, W2p, b2p)
    ms4, mv4 = _message_pass(s4, vt4, filtc, col2, row2,
                             edge_attr.reshape(E * 4), zer16, zer48)

    msg_s = ms4.transpose(1, 0, 2).reshape(N, H)
    mv3 = mv4.reshape(NCHUNK, N, 3, HC).transpose(1, 2, 0, 3).reshape(N, 3, H)
    v3 = v.transpose(0, 2, 1)

    s_new, vnew3 = _node_update(s, msg_s, mv3, v3, W3, b3, W4, b4)
    return (s_new, vnew3.transpose(0, 2, 1))


# speedup vs baseline: 12.8752x; 1.3479x over previous
import jax
import jax.numpy as jnp
from jax import lax
from jax.experimental import pallas as pl
from jax.experimental.pallas import tpu as pltpu
from jax.experimental.pallas import tpu_sc as plsc

N = 10000
E = 320000
H = 128
NCORE = 2
NSUB = 16
LANES = 16
HC = 16
NCHUNK = H // HC
CPC = NCHUNK // NCORE
B = 80
TPE = E // NSUB
NB = TPE // B
NROWS = 624
NTAIL = N - NROWS * NSUB
EB = 2560
BN = 1000


def _filter_body(rbf_ref, w1_ref, b1_ref, w2_ref, b2_ref, out_ref):
    x = rbf_ref[...]
    hmid = jnp.dot(x, w1_ref[...].T, preferred_element_type=jnp.float32)
    hmid = hmid + b1_ref[...]
    hmid = hmid * jax.nn.sigmoid(hmid)
    f = jnp.dot(hmid, w2_ref[...].T, preferred_element_type=jnp.float32)
    out_ref[...] = f + b2_ref[...]


def _filter_net(rbf, W1, b1, W2, b2):
    grid = (E // EB,)
    return pl.pallas_call(
        _filter_body,
        grid=grid,
        in_specs=[
            pl.BlockSpec((EB, 20), lambda i: (i, 0)),
            pl.BlockSpec((H, 20), lambda i: (0, 0)),
            pl.BlockSpec((1, H), lambda i: (0, 0)),
            pl.BlockSpec((2 * H, H), lambda i: (0, 0)),
            pl.BlockSpec((1, 2 * H), lambda i: (0, 0)),
        ],
        out_specs=pl.BlockSpec((EB, 2 * H), lambda i: (i, 0)),
        out_shape=jax.ShapeDtypeStruct((E, 2 * H), jnp.float32),
    )(rbf, W1, b1.reshape(1, H), W2, b2.reshape(1, 2 * H))


def _sc_body(s4, vt4, filt, col2, row2, ea, zer16, zer48,
             ms_out, mv_out,
             acc_ms, acc_mv, colv, rowv, fsb, fvb, ub0, ub1, sb, vb, msb, mvb,
             sem_in, sem_out):
    ubs = (ub0, ub1)
    core = lax.axis_index("c")
    sub = lax.axis_index("s")
    rbase = sub * NROWS
    ebase = sub * TPE

    pltpu.sync_copy(col2.at[sub], colv)
    pltpu.sync_copy(row2.at[sub], rowv)

    def issue_inputs(sl, bi, chunk):
        e0 = ebase + bi * B
        pltpu.async_copy(filt.at[pl.ds(e0, B), pl.ds(chunk * HC, HC)],
                         fsb.at[sl], sem_in.at[sl])
        pltpu.async_copy(filt.at[pl.ds(e0, B), pl.ds(H + chunk * HC, HC)],
                         fvb.at[sl], sem_in.at[sl])
        pltpu.async_copy(ea.at[pl.ds(e0 * 4, B * 4)],
                         ubs[sl].at[pl.ds(0, B * 4)], sem_in.at[sl])
        pltpu.async_copy(s4.at[chunk].at[colv.at[bi]], sb.at[sl], sem_in.at[sl])
        pltpu.async_copy(vt4.at[chunk].at[colv.at[bi]], vb.at[sl], sem_in.at[sl])

    def wait_inputs(sl):
        pltpu.make_async_copy(zer16.at[pl.ds(0, B)], fsb.at[sl], sem_in.at[sl]).wait()
        pltpu.make_async_copy(zer16.at[pl.ds(0, B)], fvb.at[sl], sem_in.at[sl]).wait()
        pltpu.make_async_copy(ea.at[pl.ds(0, B * 4)],
                              ubs[sl].at[pl.ds(0, B * 4)], sem_in.at[sl]).wait()
        pltpu.make_async_copy(zer16.at[pl.ds(0, B)], sb.at[sl], sem_in.at[sl]).wait()
        pltpu.make_async_copy(zer48.at[pl.ds(0, B)], vb.at[sl], sem_in.at[sl]).wait()

    def issue_scatter(sl, bi):
        pltpu.async_copy(msb.at[sl], acc_ms.at[rowv.at[bi]], sem_out.at[sl], add=True)
        pltpu.async_copy(mvb.at[sl], acc_mv.at[rowv.at[bi]], sem_out.at[sl], add=True)

    def wait_scatter(sl):
        pltpu.make_async_copy(zer16.at[pl.ds(0, B)], msb.at[sl], sem_out.at[sl]).wait()
        pltpu.make_async_copy(zer48.at[pl.ds(0, B)], mvb.at[sl], sem_out.at[sl]).wait()

    def compute(sl):
        uref = ubs[sl]

        def ebody(b2, carry):
            va = uref[pl.ds(b2 * 4, LANES)]
            fs = fsb[sl, b2, pl.ds(0, LANES)]
            fv = fvb[sl, b2, pl.ds(0, LANES)]
            sc_ = sb[sl, b2, pl.ds(0, LANES)]
            msb[sl, b2, pl.ds(0, LANES)] = fs * sc_
            t = fv * sc_
            for k in range(3):
                vc = vb[sl, b2, pl.ds(HC * k, LANES)]
                uk = jnp.full((LANES,), va[k + 1])
                mvb[sl, b2, pl.ds(HC * k, LANES)] = fv * vc + t * uk
            return carry

        lax.fori_loop(0, B, ebody, 0)

    def run_chunk(chunk):
        pltpu.sync_copy(zer16.at[pl.ds(rbase, NROWS)],
                        acc_ms.at[pl.ds(rbase, NROWS)])
        pltpu.sync_copy(zer48.at[pl.ds(rbase, NROWS)],
                        acc_mv.at[pl.ds(rbase, NROWS)])

        @pl.when(sub == NSUB - 1)
        def _zero_tail():
            pltpu.sync_copy(zer16.at[pl.ds(NROWS * NSUB, NTAIL)],
                            acc_ms.at[pl.ds(NROWS * NSUB, NTAIL)])
            pltpu.sync_copy(zer48.at[pl.ds(NROWS * NSUB, NTAIL)],
                            acc_mv.at[pl.ds(NROWS * NSUB, NTAIL)])

        for sl in range(2):
            pltpu.sync_copy(zer16.at[pl.ds(0, B)], msb.at[sl])
            pltpu.sync_copy(zer48.at[pl.ds(0, B)], mvb.at[sl])
        plsc.subcore_barrier()

        for sl in range(2):
            issue_scatter(sl, jnp.int32(0))
            issue_inputs(sl, jnp.int32(sl), chunk)

        def pair_body(bp, carry):
            for sl in range(2):
                bi = bp * 2 + sl
                wait_inputs(sl)
                wait_scatter(sl)
                compute(sl)
                issue_scatter(sl, bi)
                issue_inputs(sl, jnp.minimum(bi + 2, NB - 1), chunk)
            return carry

        lax.fori_loop(0, NB // 2, pair_body, 0)

        for sl in range(2):
            wait_inputs(sl)
            wait_scatter(sl)
        plsc.subcore_barrier()

        pltpu.sync_copy(acc_ms.at[pl.ds(rbase, NROWS)],
                        ms_out.at[chunk, pl.ds(rbase, NROWS)])
        pltpu.sync_copy(acc_mv.at[pl.ds(rbase, NROWS)],
                        mv_out.at[chunk, pl.ds(rbase, NROWS)])

        @pl.when(sub == NSUB - 1)
        def _flush_tail():
            pltpu.sync_copy(acc_ms.at[pl.ds(NROWS * NSUB, NTAIL)],
                            ms_out.at[chunk, pl.ds(NROWS * NSUB, NTAIL)])
            pltpu.sync_copy(acc_mv.at[pl.ds(NROWS * NSUB, NTAIL)],
                            mv_out.at[chunk, pl.ds(NROWS * NSUB, NTAIL)])

        plsc.subcore_barrier()

    for cc in range(CPC):
        @pl.when(core == 0)
        def _lo():
            run_chunk(cc)

        @pl.when(core == 1)
        def _hi():
            run_chunk(CPC + cc)


def _message_pass(s4, vt4, filt, col2, row2, ea, zer16, zer48):
    mesh = plsc.VectorSubcoreMesh(core_axis_name="c", subcore_axis_name="s",
                                  num_cores=NCORE, num_subcores=NSUB)
    f32 = jnp.float32
    return pl.kernel(
        _sc_body,
        compiler_params=pltpu.CompilerParams(use_tc_tiling_on_sc=False),
        out_type=(jax.ShapeDtypeStruct((NCHUNK, N, HC), f32),
                  jax.ShapeDtypeStruct((NCHUNK, N, 3 * HC), f32)),
        mesh=mesh,
        scratch_types=[
            pltpu.VMEM_SHARED((N, HC), f32),
            pltpu.VMEM_SHARED((N, 3 * HC), f32),
            pltpu.VMEM((NB, B), jnp.int32),
            pltpu.VMEM((NB, B), jnp.int32),
            pltpu.VMEM((2, B, HC), f32),
            pltpu.VMEM((2, B, HC), f32),
            pltpu.VMEM((B * 4 + LANES,), f32),
            pltpu.VMEM((B * 4 + LANES,), f32),
            pltpu.VMEM((2, B, HC), f32),
            pltpu.VMEM((2, B, 3 * HC), f32),
            pltpu.VMEM((2, B, HC), f32),
            pltpu.VMEM((2, B, 3 * HC), f32),
            pltpu.SemaphoreType.DMA((2,)),
            pltpu.SemaphoreType.DMA((2,)),
        ],
    )(s4, vt4, filt, col2, row2, ea, zer16, zer48)


def _update_body(s_ref, ms4_ref, mv4_ref, vt4_ref, w3_ref, b3_ref, w4_ref,
                 b4_ref, snew_ref, vnew_ref):
    s = s_ref[...]
    w3 = w3_ref[...]
    u1 = jnp.dot(s, w3[:, :H].T, preferred_element_type=jnp.float32)
    vns = []
    for c in range(NCHUNK):
        msc = ms4_ref[c]
        u1 = u1 + jnp.dot(msc, w3[:, H + HC * c:H + HC * (c + 1)].T,
                          preferred_element_type=jnp.float32)
        mvc = mv4_ref[c]
        vn = jnp.sqrt(mvc[:, :HC] ** 2 + mvc[:, HC:2 * HC] ** 2
                      + mvc[:, 2 * HC:] ** 2)
        vns.append(vn)
        u1 = u1 + jnp.dot(vn, w3[:, 2 * H + HC * c:2 * H + HC * (c + 1)].T,
                          preferred_element_type=jnp.float32)
    u1 = u1 + b3_ref[...]
    u1 = u1 * jax.nn.sigmoid(u1)
    u = jnp.dot(u1, w4_ref[...].T, preferred_element_type=jnp.float32)
    u = u + b4_ref[...]
    snew_ref[...] = s + u[:, :H]
    for c in range(NCHUNK):
        al = u[:, H + HC * c:H + HC * (c + 1)]
        be = u[:, 2 * H + HC * c:2 * H + HC * (c + 1)]
        al3 = jnp.concatenate([al, al, al], axis=1)
        be3 = jnp.concatenate([be, be, be], axis=1)
        vnew_ref[c] = al3 * vt4_ref[c] + be3 * mv4_ref[c]


def _node_update(s, ms4, mv4, vt4, W3, b3, W4, b4):
    grid = (N // BN,)
    return pl.pallas_call(
        _update_body,
        grid=grid,
        in_specs=[
            pl.BlockSpec((BN, H), lambda i: (i, 0)),
            pl.BlockSpec((NCHUNK, BN, HC), lambda i: (0, i, 0)),
            pl.BlockSpec((NCHUNK, BN, 3 * HC), lambda i: (0, i, 0)),
            pl.BlockSpec((NCHUNK, BN, 3 * HC), lambda i: (0, i, 0)),
            pl.BlockSpec((H, 3 * H), lambda i: (0, 0)),
            pl.BlockSpec((1, H), lambda i: (0, 0)),
            pl.BlockSpec((3 * H, H), lambda i: (0, 0)),
            pl.BlockSpec((1, 3 * H), lambda i: (0, 0)),
        ],
        out_specs=[
            pl.BlockSpec((BN, H), lambda i: (i, 0)),
            pl.BlockSpec((NCHUNK, BN, 3 * HC), lambda i: (0, i, 0)),
        ],
        out_shape=[
            jax.ShapeDtypeStruct((N, H), jnp.float32),
            jax.ShapeDtypeStruct((NCHUNK, N, 3 * HC), jnp.float32),
        ],
    )(s, ms4, mv4, vt4, W3, b3.reshape(1, H), W4, b4.reshape(1, 3 * H))


def kernel(s, v, edge_index, edge_attr, rbf, W1, b1, W2, b2, W3, b3, W4, b4):
    row = edge_index[0]
    col = edge_index[1]
    col2 = col.reshape(NSUB, NB, B)
    row2 = row.reshape(NSUB, NB, B)

    s4 = s.reshape(N, NCHUNK, HC).transpose(1, 0, 2)
    vt4 = v.reshape(N, NCHUNK, HC, 3).transpose(1, 0, 3, 2)
    vt4 = vt4.reshape(NCHUNK, N, 3 * HC)
    zer16 = jnp.zeros((N, HC), jnp.float32)
    zer48 = jnp.zeros((N, 3 * HC), jnp.float32)

    filt = _filter_net(rbf, W1, b1, W2, b2)
    ms4, mv4 = _message_pass(s4, vt4, filt, col2, row2,
                             edge_attr.reshape(E * 4), zer16, zer48)

    s_new, vnew4 = _node_update(s, ms4, mv4, vt4, W3, b3, W4, b4)
    v_new = (vnew4.reshape(NCHUNK, N, 3, HC).transpose(1, 0, 3, 2)
             .reshape(N, H, 3))
    return (s_new, v_new)


# speedup vs baseline: 12.8911x; 1.0012x over previous
import jax
import jax.numpy as jnp
from jax import lax
from jax.experimental import pallas as pl
from jax.experimental.pallas import tpu as pltpu
from jax.experimental.pallas import tpu_sc as plsc

N = 10000
E = 320000
H = 128
NCORE = 2
NSUB = 16
LANES = 16
HC = 16
NCHUNK = H // HC
CPC = NCHUNK // NCORE
B = 80
TPE = E // NSUB
NB = TPE // B
NROWS = 624
NTAIL = N - NROWS * NSUB
EB = 2560
BN = 1000


def _filter_body(rbf_ref, w1_ref, b1_ref, w2_ref, b2_ref, out_ref):
    x = rbf_ref[...]
    hmid = jnp.dot(x, w1_ref[...].T, preferred_element_type=jnp.float32)
    hmid = hmid + b1_ref[...]
    hmid = hmid * jax.nn.sigmoid(hmid)
    f = jnp.dot(hmid, w2_ref[...].T, preferred_element_type=jnp.float32)
    out_ref[...] = f + b2_ref[...]


def _filter_net(rbf, W1, b1, W2, b2):
    grid = (E // EB,)
    return pl.pallas_call(
        _filter_body,
        grid=grid,
        in_specs=[
            pl.BlockSpec((EB, 20), lambda i: (i, 0)),
            pl.BlockSpec((H, 20), lambda i: (0, 0)),
            pl.BlockSpec((1, H), lambda i: (0, 0)),
            pl.BlockSpec((2 * H, H), lambda i: (0, 0)),
            pl.BlockSpec((1, 2 * H), lambda i: (0, 0)),
        ],
        out_specs=pl.BlockSpec((EB, 2 * H), lambda i: (i, 0)),
        out_shape=jax.ShapeDtypeStruct((E, 2 * H), jnp.float32),
    )(rbf, W1, b1.reshape(1, H), W2, b2.reshape(1, 2 * H))


IN_BYTES_WORDS = B * (HC + HC + 4 + HC + 3 * HC)
OUT_WORDS = B * (HC + 3 * HC)


def _sc_body(s4, vt4, filt, col2, row2, ea, zer16, zer48,
             ms_out, mv_out,
             acc_ms, acc_mv, colv, rowv, fsb, fvb, ub0, ub1, sb, vb, msb, mvb,
             dummy, sem_in, sem_out):
    ubs = (ub0, ub1)
    core = lax.axis_index("c")
    sub = lax.axis_index("s")
    rbase = sub * NROWS
    ebase = sub * TPE

    pltpu.sync_copy(col2.at[sub], colv)
    pltpu.sync_copy(row2.at[sub], rowv)

    def issue_inputs(sl, bi, chunk):
        e0 = ebase + bi * B
        pltpu.async_copy(filt.at[pl.ds(e0, B), pl.ds(chunk * HC, HC)],
                         fsb.at[sl], sem_in.at[sl])
        pltpu.async_copy(filt.at[pl.ds(e0, B), pl.ds(H + chunk * HC, HC)],
                         fvb.at[sl], sem_in.at[sl])
        pltpu.async_copy(ea.at[pl.ds(e0 * 4, B * 4)],
                         ubs[sl].at[pl.ds(0, B * 4)], sem_in.at[sl])
        pltpu.async_copy(s4.at[chunk].at[colv.at[bi]], sb.at[sl], sem_in.at[sl])
        pltpu.async_copy(vt4.at[chunk].at[colv.at[bi]], vb.at[sl], sem_in.at[sl])

    def wait_inputs(sl):
        pltpu.make_async_copy(ea.at[pl.ds(0, IN_BYTES_WORDS)], dummy,
                              sem_in.at[sl]).wait()

    def issue_scatter(sl, bi):
        pltpu.async_copy(msb.at[sl], acc_ms.at[rowv.at[bi]], sem_out.at[sl], add=True)
        pltpu.async_copy(mvb.at[sl], acc_mv.at[rowv.at[bi]], sem_out.at[sl], add=True)

    def wait_scatter(sl):
        pltpu.make_async_copy(ea.at[pl.ds(0, OUT_WORDS)],
                              dummy.at[pl.ds(0, OUT_WORDS)], sem_out.at[sl]).wait()

    def compute(sl):
        uref = ubs[sl]

        def ebody(b2, carry):
            va = uref[pl.ds(b2 * 4, LANES)]
            fs = fsb[sl, b2, pl.ds(0, LANES)]
            fv = fvb[sl, b2, pl.ds(0, LANES)]
            sc_ = sb[sl, b2, pl.ds(0, LANES)]
            msb[sl, b2, pl.ds(0, LANES)] = fs * sc_
            t = fv * sc_
            for k in range(3):
                vc = vb[sl, b2, pl.ds(HC * k, LANES)]
                uk = jnp.full((LANES,), va[k + 1])
                mvb[sl, b2, pl.ds(HC * k, LANES)] = fv * vc + t * uk
            return carry

        lax.fori_loop(0, B, ebody, 0, unroll=4)

    def run_chunk(chunk):
        pltpu.sync_copy(zer16.at[pl.ds(rbase, NROWS)],
                        acc_ms.at[pl.ds(rbase, NROWS)])
        pltpu.sync_copy(zer48.at[pl.ds(rbase, NROWS)],
                        acc_mv.at[pl.ds(rbase, NROWS)])

        @pl.when(sub == NSUB - 1)
        def _zero_tail():
            pltpu.sync_copy(zer16.at[pl.ds(NROWS * NSUB, NTAIL)],
                            acc_ms.at[pl.ds(NROWS * NSUB, NTAIL)])
            pltpu.sync_copy(zer48.at[pl.ds(NROWS * NSUB, NTAIL)],
                            acc_mv.at[pl.ds(NROWS * NSUB, NTAIL)])

        for sl in range(2):
            pltpu.sync_copy(zer16.at[pl.ds(0, B)], msb.at[sl])
            pltpu.sync_copy(zer48.at[pl.ds(0, B)], mvb.at[sl])
        plsc.subcore_barrier()

        for sl in range(2):
            issue_scatter(sl, jnp.int32(0))
            issue_inputs(sl, jnp.int32(sl), chunk)

        def pair_body(bp, carry):
            for sl in range(2):
                bi = bp * 2 + sl
                wait_inputs(sl)
                wait_scatter(sl)
                compute(sl)
                issue_scatter(sl, bi)
                issue_inputs(sl, jnp.minimum(bi + 2, NB - 1), chunk)
            return carry

        lax.fori_loop(0, NB // 2, pair_body, 0)

        for sl in range(2):
            wait_inputs(sl)
            wait_scatter(sl)
        plsc.subcore_barrier()

        pltpu.sync_copy(acc_ms.at[pl.ds(rbase, NROWS)],
                        ms_out.at[chunk, pl.ds(rbase, NROWS)])
        pltpu.sync_copy(acc_mv.at[pl.ds(rbase, NROWS)],
                        mv_out.at[chunk, pl.ds(rbase, NROWS)])

        @pl.when(sub == NSUB - 1)
        def _flush_tail():
            pltpu.sync_copy(acc_ms.at[pl.ds(NROWS * NSUB, NTAIL)],
                            ms_out.at[chunk, pl.ds(NROWS * NSUB, NTAIL)])
            pltpu.sync_copy(acc_mv.at[pl.ds(NROWS * NSUB, NTAIL)],
                            mv_out.at[chunk, pl.ds(NROWS * NSUB, NTAIL)])

        plsc.subcore_barrier()

    for cc in range(CPC):
        @pl.when(core == 0)
        def _lo():
            run_chunk(cc)

        @pl.when(core == 1)
        def _hi():
            run_chunk(CPC + cc)


def _message_pass(s4, vt4, filt, col2, row2, ea, zer16, zer48):
    mesh = plsc.VectorSubcoreMesh(core_axis_name="c", subcore_axis_name="s",
                                  num_cores=NCORE, num_subcores=NSUB)
    f32 = jnp.float32
    return pl.kernel(
        _sc_body,
        compiler_params=pltpu.CompilerParams(use_tc_tiling_on_sc=False),
        out_type=(jax.ShapeDtypeStruct((NCHUNK, N, HC), f32),
                  jax.ShapeDtypeStruct((NCHUNK, N, 3 * HC), f32)),
        mesh=mesh,
        scratch_types=[
            pltpu.VMEM_SHARED((N, HC), f32),
            pltpu.VMEM_SHARED((N, 3 * HC), f32),
            pltpu.VMEM((NB, B), jnp.int32),
            pltpu.VMEM((NB, B), jnp.int32),
            pltpu.VMEM((2, B, HC), f32),
            pltpu.VMEM((2, B, HC), f32),
            pltpu.VMEM((B * 4 + LANES,), f32),
            pltpu.VMEM((B * 4 + LANES,), f32),
            pltpu.VMEM((2, B, HC), f32),
            pltpu.VMEM((2, B, 3 * HC), f32),
            pltpu.VMEM((2, B, HC), f32),
            pltpu.VMEM((2, B, 3 * HC), f32),
            pltpu.VMEM((B * (HC + HC + 4 + HC + 3 * HC),), f32),
            pltpu.SemaphoreType.DMA((2,)),
            pltpu.SemaphoreType.DMA((2,)),
        ],
    )(s4, vt4, filt, col2, row2, ea, zer16, zer48)


def _update_body(s_ref, ms4_ref, mv4_ref, vt4_ref, w3_ref, b3_ref, w4_ref,
                 b4_ref, snew_ref, vnew_ref):
    s = s_ref[...]
    w3 = w3_ref[...]
    u1 = jnp.dot(s, w3[:, :H].T, preferred_element_type=jnp.float32)
    vns = []
    for c in range(NCHUNK):
        msc = ms4_ref[c]
        u1 = u1 + jnp.dot(msc, w3[:, H + HC * c:H + HC * (c + 1)].T,
                          preferred_element_type=jnp.float32)
        mvc = mv4_ref[c]
        vn = jnp.sqrt(mvc[:, :HC] ** 2 + mvc[:, HC:2 * HC] ** 2
                      + mvc[:, 2 * HC:] ** 2)
        vns.append(vn)
        u1 = u1 + jnp.dot(vn, w3[:, 2 * H + HC * c:2 * H + HC * (c + 1)].T,
                          preferred_element_type=jnp.float32)
    u1 = u1 + b3_ref[...]
    u1 = u1 * jax.nn.sigmoid(u1)
    u = jnp.dot(u1, w4_ref[...].T, preferred_element_type=jnp.float32)
    u = u + b4_ref[...]
    snew_ref[...] = s + u[:, :H]
    for c in range(NCHUNK):
        al = u[:, H + HC * c:H + HC * (c + 1)]
        be = u[:, 2 * H + HC * c:2 * H + HC * (c + 1)]
        al3 = jnp.concatenate([al, al, al], axis=1)
        be3 = jnp.concatenate([be, be, be], axis=1)
        vnew_ref[c] = al3 * vt4_ref[c] + be3 * mv4_ref[c]


def _node_update(s, ms4, mv4, vt4, W3, b3, W4, b4):
    grid = (N // BN,)
    return pl.pallas_call(
        _update_body,
        grid=grid,
        in_specs=[
            pl.BlockSpec((BN, H), lambda i: (i, 0)),
            pl.BlockSpec((NCHUNK, BN, HC), lambda i: (0, i, 0)),
            pl.BlockSpec((NCHUNK, BN, 3 * HC), lambda i: (0, i, 0)),
            pl.BlockSpec((NCHUNK, BN, 3 * HC), lambda i: (0, i, 0)),
            pl.BlockSpec((H, 3 * H), lambda i: (0, 0)),
            pl.BlockSpec((1, H), lambda i: (0, 0)),
            pl.BlockSpec((3 * H, H), lambda i: (0, 0)),
            pl.BlockSpec((1, 3 * H), lambda i: (0, 0)),
        ],
        out_specs=[
            pl.BlockSpec((BN, H), lambda i: (i, 0)),
            pl.BlockSpec((NCHUNK, BN, 3 * HC), lambda i: (0, i, 0)),
        ],
        out_shape=[
            jax.ShapeDtypeStruct((N, H), jnp.float32),
            jax.ShapeDtypeStruct((NCHUNK, N, 3 * HC), jnp.float32),
        ],
    )(s, ms4, mv4, vt4, W3, b3.reshape(1, H), W4, b4.reshape(1, 3 * H))


def kernel(s, v, edge_index, edge_attr, rbf, W1, b1, W2, b2, W3, b3, W4, b4):
    row = edge_index[0]
    col = edge_index[1]
    col2 = col.reshape(NSUB, NB, B)
    row2 = row.reshape(NSUB, NB, B)

    s4 = s.reshape(N, NCHUNK, HC).transpose(1, 0, 2)
    vt4 = v.reshape(N, NCHUNK, HC, 3).transpose(1, 0, 3, 2)
    vt4 = vt4.reshape(NCHUNK, N, 3 * HC)
    zer16 = jnp.zeros((N, HC), jnp.float32)
    zer48 = jnp.zeros((N, 3 * HC), jnp.float32)

    filt = _filter_net(rbf, W1, b1, W2, b2)
    ms4, mv4 = _message_pass(s4, vt4, filt, col2, row2,
                             edge_attr.reshape(E * 4), zer16, zer48)

    s_new, vnew4 = _node_update(s, ms4, mv4, vt4, W3, b3, W4, b4)
    v_new = (vnew4.reshape(NCHUNK, N, 3, HC).transpose(1, 0, 3, 2)
             .reshape(N, H, 3))
    return (s_new, v_new)
